# baseline (speedup 1.0000x reference)
import jax
import jax.numpy as jnp
from jax import lax
from jax.experimental import pallas as pl
from jax.experimental.pallas import tpu as pltpu
from jax.experimental.pallas import tpu_sc as plsc

N = 10000
D = 128
NC = 2
NS = 16
NW = NC * NS
CHUNK = 128
N_PAD = 10240
ROWS_PER_TILE = N_PAD // NS
PG = 8
PAIR = 2 * PG

_MESH = plsc.VectorSubcoreMesh(
    core_axis_name="c", subcore_axis_name="s",
    num_cores=NC, num_subcores=NS)


def _make_sc_agg(nch: int):
    assert nch % PAIR == 0
    npages = nch // PG
    npairs = npages // 2
    scratch = [
        pltpu.VMEM((2 * PG, CHUNK), jnp.int32),
        pltpu.VMEM((2 * PG, CHUNK), jnp.int32),
        pltpu.VMEM((CHUNK, D), jnp.float32),
        pltpu.VMEM((CHUNK, D), jnp.float32),
        pltpu.VMEM_SHARED((N_PAD, D), jnp.float32),
        pltpu.SemaphoreType.DMA,
        pltpu.SemaphoreType.DMA,
        pltpu.SemaphoreType.DMA,
        pltpu.SemaphoreType.DMA,
    ]

    def body(h_hbm, sd_hbm, zrow_hbm, agg_out,
             sd0, sd1, rows0, rows1, agg_sh, sr0, sr1, si0, si1):
        c = lax.axis_index("c")
        s = lax.axis_index("s")
        base = s * ROWS_PER_TILE
        sds = (sd0, sd1)
        rows = (rows0, rows1)
        srs = (sr0, sr1)
        sis = (si0, si1)

        pltpu.sync_copy(zrow_hbm, rows0)
        for k in range(ROWS_PER_TILE // CHUNK):
            pltpu.sync_copy(rows0, agg_sh.at[pl.ds(base + k * CHUNK, CHUNK)])

        def wait_rows(b):
            pltpu.make_async_copy(h_hbm.at[sd0.at[0]], rows[b], srs[b]).wait()

        def wait_idx(b):
            pltpu.make_async_copy(sd_hbm.at[c, s, 0], sds[b], sis[b]).wait()

        pltpu.sync_copy(sd_hbm.at[c, s, 0], sd0)
        pltpu.async_copy(sd_hbm.at[c, s, 1], sd1, si1)
        plsc.subcore_barrier()
        pltpu.async_copy(h_hbm.at[sd0.at[0]], rows0, sr0)

        def page_pair(i, carry):
            for t in range(PAIR):
                pb, k = t // PG, t % PG
                b = t % 2
                nb = (t + 1) % 2
                if t + 1 < PAIR:
                    npb, nk = (t + 1) // PG, (t + 1) % PG
                    if t + 1 == PG:
                        wait_idx(1)
                    pltpu.async_copy(
                        h_hbm.at[sds[npb].at[nk]], rows[nb], srs[nb])
                else:
                    @pl.when(i + 1 < npairs)
                    def _():
                        wait_idx(0)
                        pltpu.async_copy(
                            h_hbm.at[sd0.at[0]], rows[nb], srs[nb])
                wait_rows(b)
                pltpu.sync_copy(rows[b], agg_sh.at[sds[pb].at[PG + k]],
                                add=True)
                if t == PG - 1:
                    @pl.when(i + 1 < npairs)
                    def _():
                        pltpu.async_copy(sd_hbm.at[c, s, 2 * i + 2], sd0, si0)
                if t == PAIR - 1:
                    @pl.when(i + 1 < npairs)
                    def _():
                        pltpu.async_copy(sd_hbm.at[c, s, 2 * i + 3], sd1, si1)
            return carry

        lax.fori_loop(0, npairs, page_pair, 0)
        plsc.subcore_barrier()

        pltpu.sync_copy(agg_sh.at[pl.ds(base, ROWS_PER_TILE)],
                        agg_out.at[c, pl.ds(base, ROWS_PER_TILE)])

    return pl.kernel(
        body,
        out_type=jax.ShapeDtypeStruct((NC, N_PAD, D), jnp.float32),
        mesh=_MESH, scratch_types=scratch)


def _make_sc_agg_simple(nch: int):
    assert nch % PG == 0
    npages = nch // PG
    scratch = [
        pltpu.VMEM((2 * PG, CHUNK), jnp.int32),
        pltpu.VMEM((CHUNK, D), jnp.float32),
        pltpu.VMEM_SHARED((N_PAD, D), jnp.float32),
    ]

    def body(h_hbm, sd_hbm, zrow_hbm, agg_out, sd_v, rows_v, agg_sh):
        c = lax.axis_index("c")
        s = lax.axis_index("s")
        base = s * ROWS_PER_TILE

        pltpu.sync_copy(zrow_hbm, rows_v)
        for k in range(ROWS_PER_TILE // CHUNK):
            pltpu.sync_copy(rows_v, agg_sh.at[pl.ds(base + k * CHUNK, CHUNK)])
        plsc.subcore_barrier()

        def page(p, carry):
            pltpu.sync_copy(sd_hbm.at[c, s, p], sd_v)
            for k in range(PG):
                pltpu.sync_copy(h_hbm.at[sd_v.at[k]], rows_v)
                pltpu.sync_copy(rows_v, agg_sh.at[sd_v.at[PG + k]], add=True)
            return carry

        lax.fori_loop(0, npages, page, 0)
        plsc.subcore_barrier()

        pltpu.sync_copy(agg_sh.at[pl.ds(base, ROWS_PER_TILE)],
                        agg_out.at[c, pl.ds(base, ROWS_PER_TILE)])

    return pl.kernel(
        body,
        out_type=jax.ShapeDtypeStruct((NC, N_PAD, D), jnp.float32),
        mesh=_MESH, scratch_types=scratch)


def _make_sc_cnt(nch: int):
    scratch = [
        pltpu.VMEM((nch, CHUNK), jnp.int32),
        pltpu.VMEM((CHUNK, D), jnp.float32),
        pltpu.VMEM_SHARED((N_PAD, D), jnp.float32),
    ]

    def body(dst_hbm, zrow_hbm, ones_hbm, cnt_out, dst_v, ones_v, cnt_sh):
        c = lax.axis_index("c")
        s = lax.axis_index("s")
        base = s * ROWS_PER_TILE

        pltpu.sync_copy(dst_hbm.at[c, s], dst_v)
        pltpu.sync_copy(zrow_hbm, ones_v)
        for k in range(ROWS_PER_TILE // CHUNK):
            pltpu.sync_copy(ones_v, cnt_sh.at[pl.ds(base + k * CHUNK, CHUNK)])
        pltpu.sync_copy(ones_hbm, ones_v)
        plsc.subcore_barrier()

        def step(j, carry):
            pltpu.sync_copy(ones_v, cnt_sh.at[dst_v.at[j]], add=True)
            return carry

        lax.fori_loop(0, nch, step, 0)
        plsc.subcore_barrier()

        pltpu.sync_copy(cnt_sh.at[pl.ds(base, ROWS_PER_TILE)],
                        cnt_out.at[c, pl.ds(base, ROWS_PER_TILE)])

    return pl.kernel(
        body,
        out_type=jax.ShapeDtypeStruct((NC, N_PAD, D), jnp.float32),
        mesh=_MESH, scratch_types=scratch)


def _dense_layer(parts, cnt_parts, h, Wl, b, Wr, g, be, with_bn: bool):
    def body(parts_ref, cnt_ref, h_ref, wl_ref, wr_ref, b_ref, g_ref,
             be_ref, o_ref):
        cnt = cnt_ref[0, :N, 0:1] + cnt_ref[1, :N, 0:1]
        inv = 1.0 / jnp.maximum(cnt, 1.0)
        mean = (parts_ref[0, :N, :] + parts_ref[1, :N, :]) * inv
        z = (jnp.dot(mean, wl_ref[...], preferred_element_type=jnp.float32)
             + jnp.dot(h_ref[...], wr_ref[...],
                       preferred_element_type=jnp.float32)
             + b_ref[...])
        if with_bn:
            mu = jnp.mean(z, axis=0, keepdims=True)
            var = jnp.mean((z - mu) ** 2, axis=0, keepdims=True)
            z = g_ref[...] * (z - mu) / jnp.sqrt(var + 1e-5) + be_ref[...]
            z = jnp.maximum(z, 0.0)
        o_ref[...] = z

    if g is None:
        g = jnp.zeros((D,), jnp.float32)
        be = jnp.zeros((D,), jnp.float32)
    return pl.pallas_call(
        body,
        out_shape=jax.ShapeDtypeStruct((N, D), jnp.float32),
    )(parts, cnt_parts, h, Wl, Wr, b.reshape(1, D), g.reshape(1, D),
      be.reshape(1, D))


def kernel(x, edge_index, W1l, b1, W1r, g1, be1, W2l, b2, W2r, g2, be2,
           W3l, b3, W3r):
    src = edge_index[0]
    dst = edge_index[1]
    E = src.shape[0]

    nchg = -(-E // (NW * CHUNK))
    nchg = -(-nchg // PAIR) * PAIR
    e_pad_g = NW * nchg * CHUNK
    npages = nchg // PG
    src_g = jnp.concatenate(
        [src, jnp.zeros((e_pad_g - E,), jnp.int32)]).reshape(
            NC, NS, npages, PG, CHUNK)
    dst_g = jnp.concatenate(
        [dst, jnp.full((e_pad_g - E,), N, jnp.int32)]).reshape(
            NC, NS, npages, PG, CHUNK)
    sd_g = jnp.concatenate([src_g, dst_g], axis=3)

    nchc = -(-E // (NW * CHUNK))
    e_pad_c = NW * nchc * CHUNK
    dst_c = jnp.concatenate(
        [dst, jnp.full((e_pad_c - E,), N, jnp.int32)]).reshape(
            NC, NS, nchc, CHUNK)

    zrow = jnp.zeros((CHUNK, D), jnp.float32)
    ones = jnp.ones((CHUNK, D), jnp.float32)

    agg_fn = _make_sc_agg(nchg)
    cnt_fn = _make_sc_cnt(nchc)

    def _jnp_agg(h):
        a = jax.ops.segment_sum(h[src], dst, num_segments=N_PAD)
        return jnp.stack([a, jnp.zeros_like(a)])

    def _jnp_cnt():
        c = jax.ops.segment_sum(jnp.ones((E,), jnp.float32), dst,
                                num_segments=N_PAD)
        c = jnp.broadcast_to(c[:, None], (N_PAD, D))
        return jnp.stack([c, jnp.zeros_like(c)])

    agg_simple_fn = _make_sc_agg_simple(nchg)
    cnt_p = cnt_fn(dst_c, zrow, ones)
    agg1 = agg_simple_fn(x, sd_g, zrow)
    h1 = _dense_layer(agg1, cnt_p, x, W1l, b1, W1r, g1, be1, with_bn=True)
    agg2 = agg_simple_fn(h1, sd_g, zrow)
    h2 = _dense_layer(agg2, cnt_p, h1, W2l, b2, W2r, g2, be2, with_bn=True)
    agg3 = agg_simple_fn(h2, sd_g, zrow)
    return _dense_layer(agg3, cnt_p, h2, W3l, b3, W3r, None, None,
                        with_bn=False)


# speedup vs baseline: 1.0962x; 1.0962x over previous
import jax
import jax.numpy as jnp
from jax import lax
from jax.experimental import pallas as pl
from jax.experimental.pallas import tpu as pltpu
from jax.experimental.pallas import tpu_sc as plsc

N = 10000
D = 128
NC = 2
NS = 16
NW = NC * NS
CHUNK = 128
N_PAD = 10240
ROWS_PER_TILE = N_PAD // NS
PG = 8
PAIR = 2 * PG

_MESH = plsc.VectorSubcoreMesh(
    core_axis_name="c", subcore_axis_name="s",
    num_cores=NC, num_subcores=NS)


def _make_sc_agg(nch: int):
    assert nch % PAIR == 0
    npages = nch // PG
    npairs = npages // 2
    scratch = [
        pltpu.VMEM((2 * PG, CHUNK), jnp.int32),
        pltpu.VMEM((2 * PG, CHUNK), jnp.int32),
        pltpu.VMEM((CHUNK, D), jnp.float32),
        pltpu.VMEM((CHUNK, D), jnp.float32),
        pltpu.VMEM_SHARED((N_PAD, D), jnp.float32),
        pltpu.SemaphoreType.DMA,
        pltpu.SemaphoreType.DMA,
        pltpu.SemaphoreType.DMA,
        pltpu.SemaphoreType.DMA,
    ]

    def body(h_hbm, sd_hbm, zrow_hbm, agg_out,
             sd0, sd1, rows0, rows1, agg_sh, sr0, sr1, si0, si1):
        c = lax.axis_index("c")
        s = lax.axis_index("s")
        base = s * ROWS_PER_TILE
        sds = (sd0, sd1)
        rows = (rows0, rows1)
        srs = (sr0, sr1)
        sis = (si0, si1)

        pltpu.sync_copy(zrow_hbm, rows0)
        for k in range(ROWS_PER_TILE // CHUNK):
            pltpu.sync_copy(rows0, agg_sh.at[pl.ds(base + k * CHUNK, CHUNK)])

        def wait_rows(b):
            pltpu.make_async_copy(h_hbm.at[sd0.at[0]], rows[b], srs[b]).wait()

        def wait_idx(b):
            pltpu.make_async_copy(sd_hbm.at[c, s, 0], sds[b], sis[b]).wait()

        pltpu.sync_copy(sd_hbm.at[c, s, 0], sd0)
        pltpu.async_copy(sd_hbm.at[c, s, 1], sd1, si1)
        plsc.subcore_barrier()
        pltpu.async_copy(h_hbm.at[sd0.at[0]], rows0, sr0)

        def page_pair(i, carry):
            for t in range(PAIR):
                pb, k = t // PG, t % PG
                b = t % 2
                nb = (t + 1) % 2
                if t + 1 < PAIR:
                    npb, nk = (t + 1) // PG, (t + 1) % PG
                    if t + 1 == PG:
                        wait_idx(1)
                    pltpu.async_copy(
                        h_hbm.at[sds[npb].at[nk]], rows[nb], srs[nb])
                else:
                    @pl.when(i + 1 < npairs)
                    def _():
                        wait_idx(0)
                        pltpu.async_copy(
                            h_hbm.at[sd0.at[0]], rows[nb], srs[nb])
                wait_rows(b)
                pltpu.sync_copy(rows[b], agg_sh.at[sds[pb].at[PG + k]],
                                add=True)
                if t == PG - 1:
                    @pl.when(i + 1 < npairs)
                    def _():
                        pltpu.async_copy(sd_hbm.at[c, s, 2 * i + 2], sd0, si0)
                if t == PAIR - 1:
                    @pl.when(i + 1 < npairs)
                    def _():
                        pltpu.async_copy(sd_hbm.at[c, s, 2 * i + 3], sd1, si1)
            return carry

        lax.fori_loop(0, npairs, page_pair, 0)
        plsc.subcore_barrier()

        pltpu.sync_copy(agg_sh.at[pl.ds(base, ROWS_PER_TILE)],
                        agg_out.at[c, pl.ds(base, ROWS_PER_TILE)])

    return pl.kernel(
        body,
        out_type=jax.ShapeDtypeStruct((NC, N_PAD, D), jnp.float32),
        mesh=_MESH, scratch_types=scratch)


def _make_sc_agg_simple(nch: int):
    assert nch % PG == 0
    npages = nch // PG
    scratch = [
        pltpu.VMEM((2 * PG, CHUNK), jnp.int32),
        pltpu.VMEM((CHUNK, D), jnp.float32),
        pltpu.VMEM_SHARED((N_PAD, D), jnp.float32),
    ]

    def body(h_hbm, sd_hbm, zrow_hbm, agg_out, sd_v, rows_v, agg_sh):
        c = lax.axis_index("c")
        s = lax.axis_index("s")
        base = s * ROWS_PER_TILE

        pltpu.sync_copy(zrow_hbm, rows_v)
        for k in range(ROWS_PER_TILE // CHUNK):
            pltpu.sync_copy(rows_v, agg_sh.at[pl.ds(base + k * CHUNK, CHUNK)])
        plsc.subcore_barrier()

        def page(p, carry):
            pltpu.sync_copy(sd_hbm.at[c, s, p], sd_v)
            for k in range(PG):
                pltpu.sync_copy(h_hbm.at[sd_v.at[k]], rows_v)
                pltpu.sync_copy(rows_v, agg_sh.at[sd_v.at[PG + k]], add=True)
            return carry

        lax.fori_loop(0, npages, page, 0)
        plsc.subcore_barrier()

        pltpu.sync_copy(agg_sh.at[pl.ds(base, ROWS_PER_TILE)],
                        agg_out.at[c, pl.ds(base, ROWS_PER_TILE)])

    return pl.kernel(
        body,
        out_type=jax.ShapeDtypeStruct((NC, N_PAD, D), jnp.float32),
        mesh=_MESH, scratch_types=scratch)


def _make_sc_agg2(nch: int):
    assert nch % (2 * PG) == 0
    npages = nch // PG
    npairs = npages // 2
    scratch = [
        pltpu.VMEM((2 * PG, CHUNK), jnp.int32),
        pltpu.VMEM((2 * PG, CHUNK), jnp.int32),
        pltpu.VMEM((CHUNK, D), jnp.float32),
        pltpu.VMEM((CHUNK, D), jnp.float32),
        pltpu.VMEM_SHARED((N_PAD, D), jnp.float32),
        pltpu.SemaphoreType.DMA,
        pltpu.SemaphoreType.DMA,
    ]

    def body(h_hbm, sd_hbm, zrow_hbm, agg_out, sd0, sd1, rows0, rows1,
             agg_sh, sr0, sr1):
        c = lax.axis_index("c")
        s = lax.axis_index("s")
        base = s * ROWS_PER_TILE
        rows = (rows0, rows1)
        srs = (sr0, sr1)
        sds = (sd0, sd1)

        pltpu.sync_copy(zrow_hbm, rows0)
        for k in range(ROWS_PER_TILE // CHUNK):
            pltpu.sync_copy(rows0, agg_sh.at[pl.ds(base + k * CHUNK, CHUNK)])
        plsc.subcore_barrier()

        def pair(i, carry):
            for pp in range(2):
                sd_v = sds[pp]
                pltpu.sync_copy(sd_hbm.at[c, s, 2 * i + pp], sd_v)
                pltpu.async_copy(h_hbm.at[sd_v.at[0]], rows[0], srs[0])
                for k in range(PG):
                    b = k % 2
                    if k + 1 < PG:
                        pltpu.async_copy(
                            h_hbm.at[sd_v.at[k + 1]], rows[1 - b],
                            srs[1 - b])
                    pltpu.make_async_copy(
                        h_hbm.at[sd_v.at[k]], rows[b], srs[b]).wait()
                    pltpu.sync_copy(rows[b], agg_sh.at[sd_v.at[PG + k]],
                                    add=True)
            return carry

        lax.fori_loop(0, npairs, pair, 0)
        plsc.subcore_barrier()

        pltpu.sync_copy(agg_sh.at[pl.ds(base, ROWS_PER_TILE)],
                        agg_out.at[c, pl.ds(base, ROWS_PER_TILE)])

    return pl.kernel(
        body,
        out_type=jax.ShapeDtypeStruct((NC, N_PAD, D), jnp.float32),
        mesh=_MESH, scratch_types=scratch)


def _make_sc_cnt(nch: int):
    scratch = [
        pltpu.VMEM((nch, CHUNK), jnp.int32),
        pltpu.VMEM((CHUNK, D), jnp.float32),
        pltpu.VMEM_SHARED((N_PAD, D), jnp.float32),
    ]

    def body(dst_hbm, zrow_hbm, ones_hbm, cnt_out, dst_v, ones_v, cnt_sh):
        c = lax.axis_index("c")
        s = lax.axis_index("s")
        base = s * ROWS_PER_TILE

        pltpu.sync_copy(dst_hbm.at[c, s], dst_v)
        pltpu.sync_copy(zrow_hbm, ones_v)
        for k in range(ROWS_PER_TILE // CHUNK):
            pltpu.sync_copy(ones_v, cnt_sh.at[pl.ds(base + k * CHUNK, CHUNK)])
        pltpu.sync_copy(ones_hbm, ones_v)
        plsc.subcore_barrier()

        def step(j, carry):
            pltpu.sync_copy(ones_v, cnt_sh.at[dst_v.at[j]], add=True)
            return carry

        lax.fori_loop(0, nch, step, 0)
        plsc.subcore_barrier()

        pltpu.sync_copy(cnt_sh.at[pl.ds(base, ROWS_PER_TILE)],
                        cnt_out.at[c, pl.ds(base, ROWS_PER_TILE)])

    return pl.kernel(
        body,
        out_type=jax.ShapeDtypeStruct((NC, N_PAD, D), jnp.float32),
        mesh=_MESH, scratch_types=scratch)


def _dense_layer(parts, cnt_parts, h, Wl, b, Wr, g, be, with_bn: bool):
    def body(parts_ref, cnt_ref, h_ref, wl_ref, wr_ref, b_ref, g_ref,
             be_ref, o_ref):
        cnt = cnt_ref[0, :N, 0:1] + cnt_ref[1, :N, 0:1]
        inv = 1.0 / jnp.maximum(cnt, 1.0)
        mean = (parts_ref[0, :N, :] + parts_ref[1, :N, :]) * inv
        z = (jnp.dot(mean, wl_ref[...], preferred_element_type=jnp.float32)
             + jnp.dot(h_ref[...], wr_ref[...],
                       preferred_element_type=jnp.float32)
             + b_ref[...])
        if with_bn:
            mu = jnp.mean(z, axis=0, keepdims=True)
            var = jnp.mean((z - mu) ** 2, axis=0, keepdims=True)
            z = g_ref[...] * (z - mu) / jnp.sqrt(var + 1e-5) + be_ref[...]
            z = jnp.maximum(z, 0.0)
        o_ref[...] = z

    if g is None:
        g = jnp.zeros((D,), jnp.float32)
        be = jnp.zeros((D,), jnp.float32)
    return pl.pallas_call(
        body,
        out_shape=jax.ShapeDtypeStruct((N, D), jnp.float32),
    )(parts, cnt_parts, h, Wl, Wr, b.reshape(1, D), g.reshape(1, D),
      be.reshape(1, D))


def kernel(x, edge_index, W1l, b1, W1r, g1, be1, W2l, b2, W2r, g2, be2,
           W3l, b3, W3r):
    src = edge_index[0]
    dst = edge_index[1]
    E = src.shape[0]

    nchg = -(-E // (NW * CHUNK))
    nchg = -(-nchg // PAIR) * PAIR
    e_pad_g = NW * nchg * CHUNK
    npages = nchg // PG
    src_g = jnp.concatenate(
        [src, jnp.zeros((e_pad_g - E,), jnp.int32)]).reshape(
            NC, NS, npages, PG, CHUNK)
    dst_g = jnp.concatenate(
        [dst, jnp.full((e_pad_g - E,), N, jnp.int32)]).reshape(
            NC, NS, npages, PG, CHUNK)
    sd_g = jnp.concatenate([src_g, dst_g], axis=3)

    nchc = -(-E // (NW * CHUNK))
    e_pad_c = NW * nchc * CHUNK
    dst_c = jnp.concatenate(
        [dst, jnp.full((e_pad_c - E,), N, jnp.int32)]).reshape(
            NC, NS, nchc, CHUNK)

    zrow = jnp.zeros((CHUNK, D), jnp.float32)
    ones = jnp.ones((CHUNK, D), jnp.float32)

    agg_fn = _make_sc_agg(nchg)
    cnt_fn = _make_sc_cnt(nchc)

    def _jnp_agg(h):
        a = jax.ops.segment_sum(h[src], dst, num_segments=N_PAD)
        return jnp.stack([a, jnp.zeros_like(a)])

    def _jnp_cnt():
        c = jax.ops.segment_sum(jnp.ones((E,), jnp.float32), dst,
                                num_segments=N_PAD)
        c = jnp.broadcast_to(c[:, None], (N_PAD, D))
        return jnp.stack([c, jnp.zeros_like(c)])

    agg2_fn = _make_sc_agg2(nchg)
    cnt_p = cnt_fn(dst_c, zrow, ones)
    agg1 = agg2_fn(x, sd_g, zrow)
    h1 = _dense_layer(agg1, cnt_p, x, W1l, b1, W1r, g1, be1, with_bn=True)
    agg2 = agg2_fn(h1, sd_g, zrow)
    h2 = _dense_layer(agg2, cnt_p, h1, W2l, b2, W2r, g2, be2, with_bn=True)
    agg3 = agg2_fn(h2, sd_g, zrow)
    return _dense_layer(agg3, cnt_p, h2, W3l, b3, W3r, None, None,
                        with_bn=False)


# speedup vs baseline: 1.2232x; 1.1158x over previous
import jax
import jax.numpy as jnp
from jax import lax
from jax.experimental import pallas as pl
from jax.experimental.pallas import tpu as pltpu
from jax.experimental.pallas import tpu_sc as plsc

N = 10000
D = 128
NC = 2
NS = 16
NW = NC * NS
CHUNK = 128
N_PAD = 10240
ROWS_PER_TILE = N_PAD // NS
PG = 8
PAIR = 2 * PG

_MESH = plsc.VectorSubcoreMesh(
    core_axis_name="c", subcore_axis_name="s",
    num_cores=NC, num_subcores=NS)


def _make_sc_agg(nch: int):
    assert nch % PAIR == 0
    npages = nch // PG
    npairs = npages // 2
    scratch = [
        pltpu.VMEM((2 * PG, CHUNK), jnp.int32),
        pltpu.VMEM((2 * PG, CHUNK), jnp.int32),
        pltpu.VMEM((CHUNK, D), jnp.float32),
        pltpu.VMEM((CHUNK, D), jnp.float32),
        pltpu.VMEM_SHARED((N_PAD, D), jnp.float32),
        pltpu.SemaphoreType.DMA,
        pltpu.SemaphoreType.DMA,
        pltpu.SemaphoreType.DMA,
        pltpu.SemaphoreType.DMA,
    ]

    def body(h_hbm, sd_hbm, zrow_hbm, agg_out,
             sd0, sd1, rows0, rows1, agg_sh, sr0, sr1, si0, si1):
        c = lax.axis_index("c")
        s = lax.axis_index("s")
        base = s * ROWS_PER_TILE
        sds = (sd0, sd1)
        rows = (rows0, rows1)
        srs = (sr0, sr1)
        sis = (si0, si1)

        pltpu.sync_copy(zrow_hbm, rows0)
        for k in range(ROWS_PER_TILE // CHUNK):
            pltpu.sync_copy(rows0, agg_sh.at[pl.ds(base + k * CHUNK, CHUNK)])

        def wait_rows(b):
            pltpu.make_async_copy(h_hbm.at[sd0.at[0]], rows[b], srs[b]).wait()

        def wait_idx(b):
            pltpu.make_async_copy(sd_hbm.at[c, s, 0], sds[b], sis[b]).wait()

        pltpu.sync_copy(sd_hbm.at[c, s, 0], sd0)
        pltpu.async_copy(sd_hbm.at[c, s, 1], sd1, si1)
        plsc.subcore_barrier()
        pltpu.async_copy(h_hbm.at[sd0.at[0]], rows0, sr0)

        def page_pair(i, carry):
            for t in range(PAIR):
                pb, k = t // PG, t % PG
                b = t % 2
                nb = (t + 1) % 2
                if t + 1 < PAIR:
                    npb, nk = (t + 1) // PG, (t + 1) % PG
                    if t + 1 == PG:
                        wait_idx(1)
                    pltpu.async_copy(
                        h_hbm.at[sds[npb].at[nk]], rows[nb], srs[nb])
                else:
                    @pl.when(i + 1 < npairs)
                    def _():
                        wait_idx(0)
                        pltpu.async_copy(
                            h_hbm.at[sd0.at[0]], rows[nb], srs[nb])
                wait_rows(b)
                pltpu.sync_copy(rows[b], agg_sh.at[sds[pb].at[PG + k]],
                                add=True)
                if t == PG - 1:
                    @pl.when(i + 1 < npairs)
                    def _():
                        pltpu.async_copy(sd_hbm.at[c, s, 2 * i + 2], sd0, si0)
                if t == PAIR - 1:
                    @pl.when(i + 1 < npairs)
                    def _():
                        pltpu.async_copy(sd_hbm.at[c, s, 2 * i + 3], sd1, si1)
            return carry

        lax.fori_loop(0, npairs, page_pair, 0)
        plsc.subcore_barrier()

        pltpu.sync_copy(agg_sh.at[pl.ds(base, ROWS_PER_TILE)],
                        agg_out.at[c, pl.ds(base, ROWS_PER_TILE)])

    return pl.kernel(
        body,
        out_type=jax.ShapeDtypeStruct((NC, N_PAD, D), jnp.float32),
        mesh=_MESH, scratch_types=scratch)


def _make_sc_agg_simple(nch: int):
    assert nch % PG == 0
    npages = nch // PG
    scratch = [
        pltpu.VMEM((2 * PG, CHUNK), jnp.int32),
        pltpu.VMEM((CHUNK, D), jnp.float32),
        pltpu.VMEM_SHARED((N_PAD, D), jnp.float32),
    ]

    def body(h_hbm, sd_hbm, zrow_hbm, agg_out, sd_v, rows_v, agg_sh):
        c = lax.axis_index("c")
        s = lax.axis_index("s")
        base = s * ROWS_PER_TILE

        pltpu.sync_copy(zrow_hbm, rows_v)
        for k in range(ROWS_PER_TILE // CHUNK):
            pltpu.sync_copy(rows_v, agg_sh.at[pl.ds(base + k * CHUNK, CHUNK)])
        plsc.subcore_barrier()

        def page(p, carry):
            pltpu.sync_copy(sd_hbm.at[c, s, p], sd_v)
            for k in range(PG):
                pltpu.sync_copy(h_hbm.at[sd_v.at[k]], rows_v)
                pltpu.sync_copy(rows_v, agg_sh.at[sd_v.at[PG + k]], add=True)
            return carry

        lax.fori_loop(0, npages, page, 0)
        plsc.subcore_barrier()

        pltpu.sync_copy(agg_sh.at[pl.ds(base, ROWS_PER_TILE)],
                        agg_out.at[c, pl.ds(base, ROWS_PER_TILE)])

    return pl.kernel(
        body,
        out_type=jax.ShapeDtypeStruct((NC, N_PAD, D), jnp.float32),
        mesh=_MESH, scratch_types=scratch)


def _make_sc_agg2(nch: int):
    assert nch % (2 * PG) == 0
    npages = nch // PG
    npairs = npages // 2
    scratch = [
        pltpu.VMEM((2 * PG, CHUNK), jnp.int32),
        pltpu.VMEM((2 * PG, CHUNK), jnp.int32),
        pltpu.VMEM((CHUNK, D), jnp.float32),
        pltpu.VMEM((CHUNK, D), jnp.float32),
        pltpu.VMEM_SHARED((N_PAD, D), jnp.float32),
        pltpu.SemaphoreType.DMA,
        pltpu.SemaphoreType.DMA,
    ]

    def body(h_hbm, sd_hbm, zrow_hbm, agg_out, sd0, sd1, rows0, rows1,
             agg_sh, sr0, sr1):
        c = lax.axis_index("c")
        s = lax.axis_index("s")
        base = s * ROWS_PER_TILE
        rows = (rows0, rows1)
        srs = (sr0, sr1)
        sds = (sd0, sd1)

        pltpu.sync_copy(zrow_hbm, rows0)
        for k in range(ROWS_PER_TILE // CHUNK):
            pltpu.sync_copy(rows0, agg_sh.at[pl.ds(base + k * CHUNK, CHUNK)])
        plsc.subcore_barrier()

        def pair(i, carry):
            for pp in range(2):
                sd_v = sds[pp]
                pltpu.sync_copy(sd_hbm.at[c, s, 2 * i + pp], sd_v)
                pltpu.async_copy(h_hbm.at[sd_v.at[0]], rows[0], srs[0])
                for k in range(PG):
                    b = k % 2
                    if k + 1 < PG:
                        pltpu.async_copy(
                            h_hbm.at[sd_v.at[k + 1]], rows[1 - b],
                            srs[1 - b])
                    pltpu.make_async_copy(
                        h_hbm.at[sd_v.at[k]], rows[b], srs[b]).wait()
                    pltpu.sync_copy(rows[b], agg_sh.at[sd_v.at[PG + k]],
                                    add=True)
            return carry

        lax.fori_loop(0, npairs, pair, 0)
        plsc.subcore_barrier()

        pltpu.sync_copy(agg_sh.at[pl.ds(base, ROWS_PER_TILE)],
                        agg_out.at[c, pl.ds(base, ROWS_PER_TILE)])

    return pl.kernel(
        body,
        out_type=jax.ShapeDtypeStruct((NC, N_PAD, D), jnp.float32),
        mesh=_MESH, scratch_types=scratch)


def _make_sc_cnt(nch: int):
    scratch = [
        pltpu.VMEM((nch, CHUNK), jnp.int32),
        pltpu.VMEM((CHUNK, D), jnp.float32),
        pltpu.VMEM_SHARED((N_PAD, D), jnp.float32),
    ]

    def body(dst_hbm, zrow_hbm, ones_hbm, cnt_out, dst_v, ones_v, cnt_sh):
        c = lax.axis_index("c")
        s = lax.axis_index("s")
        base = s * ROWS_PER_TILE

        pltpu.sync_copy(dst_hbm.at[c, s], dst_v)
        pltpu.sync_copy(zrow_hbm, ones_v)
        for k in range(ROWS_PER_TILE // CHUNK):
            pltpu.sync_copy(ones_v, cnt_sh.at[pl.ds(base + k * CHUNK, CHUNK)])
        pltpu.sync_copy(ones_hbm, ones_v)
        plsc.subcore_barrier()

        def step(j, carry):
            pltpu.sync_copy(ones_v, cnt_sh.at[dst_v.at[j]], add=True)
            return carry

        lax.fori_loop(0, nch, step, 0)
        plsc.subcore_barrier()

        pltpu.sync_copy(cnt_sh.at[pl.ds(base, ROWS_PER_TILE)],
                        cnt_out.at[c, pl.ds(base, ROWS_PER_TILE)])

    return pl.kernel(
        body,
        out_type=jax.ShapeDtypeStruct((NC, N_PAD, D), jnp.float32),
        mesh=_MESH, scratch_types=scratch)


def _dense_layer(parts, cnt_parts, h, Wl, b, Wr, g, be, with_bn: bool):
    def body(parts_ref, cnt_ref, h_ref, wl_ref, wr_ref, b_ref, g_ref,
             be_ref, o_ref):
        cnt = cnt_ref[0, :N, 0:1] + cnt_ref[1, :N, 0:1]
        inv = 1.0 / jnp.maximum(cnt, 1.0)
        mean = (parts_ref[0, :N, :] + parts_ref[1, :N, :]) * inv
        z = (jnp.dot(mean, wl_ref[...], preferred_element_type=jnp.float32)
             + jnp.dot(h_ref[...], wr_ref[...],
                       preferred_element_type=jnp.float32)
             + b_ref[...])
        if with_bn:
            mu = jnp.mean(z, axis=0, keepdims=True)
            var = jnp.mean((z - mu) ** 2, axis=0, keepdims=True)
            z = g_ref[...] * (z - mu) / jnp.sqrt(var + 1e-5) + be_ref[...]
            z = jnp.maximum(z, 0.0)
        o_ref[...] = z

    if g is None:
        g = jnp.zeros((D,), jnp.float32)
        be = jnp.zeros((D,), jnp.float32)
    return pl.pallas_call(
        body,
        out_shape=jax.ShapeDtypeStruct((N, D), jnp.float32),
    )(parts, cnt_parts, h, Wl, Wr, b.reshape(1, D), g.reshape(1, D),
      be.reshape(1, D))


def kernel(x, edge_index, W1l, b1, W1r, g1, be1, W2l, b2, W2r, g2, be2,
           W3l, b3, W3r):
    src = edge_index[0]
    dst = edge_index[1]
    E = src.shape[0]

    nchg = -(-E // (NW * CHUNK))
    nchg = -(-nchg // PAIR) * PAIR
    e_pad_g = NW * nchg * CHUNK
    npages = nchg // PG
    M = e_pad_g // CHUNK
    perm = jnp.arange(e_pad_g, dtype=jnp.int32).reshape(CHUNK, M).T.ravel()
    src_g = jnp.concatenate(
        [src, jnp.zeros((e_pad_g - E,), jnp.int32)])[perm].reshape(
            NC, NS, npages, PG, CHUNK)
    dst_g = jnp.concatenate(
        [dst, jnp.full((e_pad_g - E,), N, jnp.int32)])[perm].reshape(
            NC, NS, npages, PG, CHUNK)
    sd_g = jnp.concatenate([src_g, dst_g], axis=3)

    nchc = -(-E // (NW * CHUNK))
    e_pad_c = NW * nchc * CHUNK
    Mc = e_pad_c // CHUNK
    perm_c = jnp.arange(e_pad_c, dtype=jnp.int32).reshape(CHUNK, Mc).T.ravel()
    dst_c = jnp.concatenate(
        [dst, jnp.full((e_pad_c - E,), N, jnp.int32)])[perm_c].reshape(
            NC, NS, nchc, CHUNK)

    zrow = jnp.zeros((CHUNK, D), jnp.float32)
    ones = jnp.ones((CHUNK, D), jnp.float32)

    agg_fn = _make_sc_agg(nchg)
    cnt_fn = _make_sc_cnt(nchc)

    def _jnp_agg(h):
        a = jax.ops.segment_sum(h[src], dst, num_segments=N_PAD)
        return jnp.stack([a, jnp.zeros_like(a)])

    def _jnp_cnt():
        c = jax.ops.segment_sum(jnp.ones((E,), jnp.float32), dst,
                                num_segments=N_PAD)
        c = jnp.broadcast_to(c[:, None], (N_PAD, D))
        return jnp.stack([c, jnp.zeros_like(c)])

    agg2_fn = _make_sc_agg2(nchg)
    cnt_p = cnt_fn(dst_c, zrow, ones)
    agg1 = agg2_fn(x, sd_g, zrow)
    h1 = _dense_layer(agg1, cnt_p, x, W1l, b1, W1r, g1, be1, with_bn=True)
    agg2 = agg2_fn(h1, sd_g, zrow)
    h2 = _dense_layer(agg2, cnt_p, h1, W2l, b2, W2r, g2, be2, with_bn=True)
    agg3 = agg2_fn(h2, sd_g, zrow)
    return _dense_layer(agg3, cnt_p, h2, W3l, b3, W3r, None, None,
                        with_bn=False)


# speedup vs baseline: 1.2232x; 1.0000x over previous
import jax
import jax.numpy as jnp
from jax import lax
from jax.experimental import pallas as pl
from jax.experimental.pallas import tpu as pltpu
from jax.experimental.pallas import tpu_sc as plsc

N = 10000
D = 128
NC = 2
NS = 16
NW = NC * NS
CHUNK = 128
N_PAD = 10240
ROWS_PER_TILE = N_PAD // NS
PG = 8
PAIR = 2 * PG

_MESH = plsc.VectorSubcoreMesh(
    core_axis_name="c", subcore_axis_name="s",
    num_cores=NC, num_subcores=NS)


def _make_sc_agg(nch: int):
    assert nch % PAIR == 0
    npages = nch // PG
    npairs = npages // 2
    scratch = [
        pltpu.VMEM((2 * PG, CHUNK), jnp.int32),
        pltpu.VMEM((2 * PG, CHUNK), jnp.int32),
        pltpu.VMEM((CHUNK, D), jnp.float32),
        pltpu.VMEM((CHUNK, D), jnp.float32),
        pltpu.VMEM_SHARED((N_PAD, D), jnp.float32),
        pltpu.SemaphoreType.DMA,
        pltpu.SemaphoreType.DMA,
        pltpu.SemaphoreType.DMA,
        pltpu.SemaphoreType.DMA,
    ]

    def body(h_hbm, sd_hbm, zrow_hbm, agg_out,
             sd0, sd1, rows0, rows1, agg_sh, sr0, sr1, si0, si1):
        c = lax.axis_index("c")
        s = lax.axis_index("s")
        base = s * ROWS_PER_TILE
        sds = (sd0, sd1)
        rows = (rows0, rows1)
        srs = (sr0, sr1)
        sis = (si0, si1)

        pltpu.sync_copy(zrow_hbm, rows0)
        for k in range(ROWS_PER_TILE // CHUNK):
            pltpu.sync_copy(rows0, agg_sh.at[pl.ds(base + k * CHUNK, CHUNK)])

        def wait_idx(b):
            pltpu.make_async_copy(sd_hbm.at[c, s, 0], sds[b], sis[b]).wait()

        pltpu.sync_copy(sd_hbm.at[c, s, 0], sd0)
        pltpu.async_copy(sd_hbm.at[c, s, 1], sd1, si1)
        plsc.subcore_barrier()
        pltpu.async_copy(h_hbm.at[sd0.at[0]], rows0, sr0)

        def page_pair(i, carry):
            for t in range(PAIR):
                pb, k = t // PG, t % PG
                b = t % 2
                nb = (t + 1) % 2
                if t + 1 < PAIR:
                    npb, nk = (t + 1) // PG, (t + 1) % PG
                    if t + 1 == PG:
                        wait_idx(1)
                    pltpu.async_copy(
                        h_hbm.at[sds[npb].at[nk]], rows[nb], srs[nb])
                else:
                    @pl.when(i + 1 < npairs)
                    def _():
                        wait_idx(0)
                        pltpu.async_copy(
                            h_hbm.at[sd0.at[0]], rows[nb], srs[nb])
                pltpu.make_async_copy(
                    h_hbm.at[sds[pb].at[k]], rows[b], srs[b]).wait()
                pltpu.sync_copy(rows[b], agg_sh.at[sds[pb].at[PG + k]],
                                add=True)
                if t == PG - 1:
                    @pl.when(i + 1 < npairs)
                    def _():
                        pltpu.async_copy(sd_hbm.at[c, s, 2 * i + 2], sd0, si0)
                if t == PAIR - 1:
                    @pl.when(i + 1 < npairs)
                    def _():
                        pltpu.async_copy(sd_hbm.at[c, s, 2 * i + 3], sd1, si1)
            return carry

        lax.fori_loop(0, npairs, page_pair, 0)
        plsc.subcore_barrier()

        pltpu.sync_copy(agg_sh.at[pl.ds(base, ROWS_PER_TILE)],
                        agg_out.at[c, pl.ds(base, ROWS_PER_TILE)])

    return pl.kernel(
        body,
        out_type=jax.ShapeDtypeStruct((NC, N_PAD, D), jnp.float32),
        mesh=_MESH, scratch_types=scratch)


def _make_sc_agg_simple(nch: int):
    assert nch % PG == 0
    npages = nch // PG
    scratch = [
        pltpu.VMEM((2 * PG, CHUNK), jnp.int32),
        pltpu.VMEM((CHUNK, D), jnp.float32),
        pltpu.VMEM_SHARED((N_PAD, D), jnp.float32),
    ]

    def body(h_hbm, sd_hbm, zrow_hbm, agg_out, sd_v, rows_v, agg_sh):
        c = lax.axis_index("c")
        s = lax.axis_index("s")
        base = s * ROWS_PER_TILE

        pltpu.sync_copy(zrow_hbm, rows_v)
        for k in range(ROWS_PER_TILE // CHUNK):
            pltpu.sync_copy(rows_v, agg_sh.at[pl.ds(base + k * CHUNK, CHUNK)])
        plsc.subcore_barrier()

        def page(p, carry):
            pltpu.sync_copy(sd_hbm.at[c, s, p], sd_v)
            for k in range(PG):
                pltpu.sync_copy(h_hbm.at[sd_v.at[k]], rows_v)
                pltpu.sync_copy(rows_v, agg_sh.at[sd_v.at[PG + k]], add=True)
            return carry

        lax.fori_loop(0, npages, page, 0)
        plsc.subcore_barrier()

        pltpu.sync_copy(agg_sh.at[pl.ds(base, ROWS_PER_TILE)],
                        agg_out.at[c, pl.ds(base, ROWS_PER_TILE)])

    return pl.kernel(
        body,
        out_type=jax.ShapeDtypeStruct((NC, N_PAD, D), jnp.float32),
        mesh=_MESH, scratch_types=scratch)


def _make_sc_agg2(nch: int):
    assert nch % (2 * PG) == 0
    npages = nch // PG
    npairs = npages // 2
    scratch = [
        pltpu.VMEM((2 * PG, CHUNK), jnp.int32),
        pltpu.VMEM((2 * PG, CHUNK), jnp.int32),
        pltpu.VMEM((CHUNK, D), jnp.float32),
        pltpu.VMEM((CHUNK, D), jnp.float32),
        pltpu.VMEM_SHARED((N_PAD, D), jnp.float32),
        pltpu.SemaphoreType.DMA,
        pltpu.SemaphoreType.DMA,
    ]

    def body(h_hbm, sd_hbm, zrow_hbm, agg_out, sd0, sd1, rows0, rows1,
             agg_sh, sr0, sr1):
        c = lax.axis_index("c")
        s = lax.axis_index("s")
        base = s * ROWS_PER_TILE
        rows = (rows0, rows1)
        srs = (sr0, sr1)
        sds = (sd0, sd1)

        pltpu.sync_copy(zrow_hbm, rows0)
        for k in range(ROWS_PER_TILE // CHUNK):
            pltpu.sync_copy(rows0, agg_sh.at[pl.ds(base + k * CHUNK, CHUNK)])
        plsc.subcore_barrier()

        def pair(i, carry):
            for pp in range(2):
                sd_v = sds[pp]
                pltpu.sync_copy(sd_hbm.at[c, s, 2 * i + pp], sd_v)
                pltpu.async_copy(h_hbm.at[sd_v.at[0]], rows[0], srs[0])
                for k in range(PG):
                    b = k % 2
                    if k + 1 < PG:
                        pltpu.async_copy(
                            h_hbm.at[sd_v.at[k + 1]], rows[1 - b],
                            srs[1 - b])
                    pltpu.make_async_copy(
                        h_hbm.at[sd_v.at[k]], rows[b], srs[b]).wait()
                    pltpu.sync_copy(rows[b], agg_sh.at[sd_v.at[PG + k]],
                                    add=True)
            return carry

        lax.fori_loop(0, npairs, pair, 0)
        plsc.subcore_barrier()

        pltpu.sync_copy(agg_sh.at[pl.ds(base, ROWS_PER_TILE)],
                        agg_out.at[c, pl.ds(base, ROWS_PER_TILE)])

    return pl.kernel(
        body,
        out_type=jax.ShapeDtypeStruct((NC, N_PAD, D), jnp.float32),
        mesh=_MESH, scratch_types=scratch)


def _make_sc_cnt(nch: int):
    scratch = [
        pltpu.VMEM((nch, CHUNK), jnp.int32),
        pltpu.VMEM((CHUNK, D), jnp.float32),
        pltpu.VMEM_SHARED((N_PAD, D), jnp.float32),
    ]

    def body(dst_hbm, zrow_hbm, ones_hbm, cnt_out, dst_v, ones_v, cnt_sh):
        c = lax.axis_index("c")
        s = lax.axis_index("s")
        base = s * ROWS_PER_TILE

        pltpu.sync_copy(dst_hbm.at[c, s], dst_v)
        pltpu.sync_copy(zrow_hbm, ones_v)
        for k in range(ROWS_PER_TILE // CHUNK):
            pltpu.sync_copy(ones_v, cnt_sh.at[pl.ds(base + k * CHUNK, CHUNK)])
        pltpu.sync_copy(ones_hbm, ones_v)
        plsc.subcore_barrier()

        def step(j, carry):
            pltpu.sync_copy(ones_v, cnt_sh.at[dst_v.at[j]], add=True)
            return carry

        lax.fori_loop(0, nch, step, 0)
        plsc.subcore_barrier()

        pltpu.sync_copy(cnt_sh.at[pl.ds(base, ROWS_PER_TILE)],
                        cnt_out.at[c, pl.ds(base, ROWS_PER_TILE)])

    return pl.kernel(
        body,
        out_type=jax.ShapeDtypeStruct((NC, N_PAD, D), jnp.float32),
        mesh=_MESH, scratch_types=scratch)


def _dense_layer(parts, cnt_parts, h, Wl, b, Wr, g, be, with_bn: bool):
    def body(parts_ref, cnt_ref, h_ref, wl_ref, wr_ref, b_ref, g_ref,
             be_ref, o_ref):
        cnt = cnt_ref[0, :N, 0:1] + cnt_ref[1, :N, 0:1]
        inv = 1.0 / jnp.maximum(cnt, 1.0)
        mean = (parts_ref[0, :N, :] + parts_ref[1, :N, :]) * inv
        z = (jnp.dot(mean, wl_ref[...], preferred_element_type=jnp.float32)
             + jnp.dot(h_ref[...], wr_ref[...],
                       preferred_element_type=jnp.float32)
             + b_ref[...])
        if with_bn:
            mu = jnp.mean(z, axis=0, keepdims=True)
            var = jnp.mean((z - mu) ** 2, axis=0, keepdims=True)
            z = g_ref[...] * (z - mu) / jnp.sqrt(var + 1e-5) + be_ref[...]
            z = jnp.maximum(z, 0.0)
        o_ref[...] = z

    if g is None:
        g = jnp.zeros((D,), jnp.float32)
        be = jnp.zeros((D,), jnp.float32)
    return pl.pallas_call(
        body,
        out_shape=jax.ShapeDtypeStruct((N, D), jnp.float32),
    )(parts, cnt_parts, h, Wl, Wr, b.reshape(1, D), g.reshape(1, D),
      be.reshape(1, D))


def kernel(x, edge_index, W1l, b1, W1r, g1, be1, W2l, b2, W2r, g2, be2,
           W3l, b3, W3r):
    src = edge_index[0]
    dst = edge_index[1]
    E = src.shape[0]

    nchg = -(-E // (NW * CHUNK))
    nchg = -(-nchg // PAIR) * PAIR
    e_pad_g = NW * nchg * CHUNK
    npages = nchg // PG
    M = e_pad_g // CHUNK
    perm = jnp.arange(e_pad_g, dtype=jnp.int32).reshape(CHUNK, M).T.ravel()
    src_g = jnp.concatenate(
        [src, jnp.zeros((e_pad_g - E,), jnp.int32)])[perm].reshape(
            NC, NS, npages, PG, CHUNK)
    dst_g = jnp.concatenate(
        [dst, jnp.full((e_pad_g - E,), N, jnp.int32)])[perm].reshape(
            NC, NS, npages, PG, CHUNK)
    sd_g = jnp.concatenate([src_g, dst_g], axis=3)

    nchc = -(-E // (NW * CHUNK))
    e_pad_c = NW * nchc * CHUNK
    Mc = e_pad_c // CHUNK
    perm_c = jnp.arange(e_pad_c, dtype=jnp.int32).reshape(CHUNK, Mc).T.ravel()
    dst_c = jnp.concatenate(
        [dst, jnp.full((e_pad_c - E,), N, jnp.int32)])[perm_c].reshape(
            NC, NS, nchc, CHUNK)

    zrow = jnp.zeros((CHUNK, D), jnp.float32)
    ones = jnp.ones((CHUNK, D), jnp.float32)

    agg_fn = _make_sc_agg(nchg)
    cnt_fn = _make_sc_cnt(nchc)

    cnt_p = cnt_fn(dst_c, zrow, ones)
    agg1 = agg_fn(x, sd_g, zrow)
    h1 = _dense_layer(agg1, cnt_p, x, W1l, b1, W1r, g1, be1, with_bn=True)
    agg2 = agg_fn(h1, sd_g, zrow)
    h2 = _dense_layer(agg2, cnt_p, h1, W2l, b2, W2r, g2, be2, with_bn=True)
    agg3 = agg_fn(h2, sd_g, zrow)
    return _dense_layer(agg3, cnt_p, h2, W3l, b3, W3r, None, None,
                        with_bn=False)
